# baseline (speedup 1.0000x reference)
import functools

import jax
import jax.numpy as jnp
from jax import lax
from jax.experimental import pallas as pl
from jax.experimental.pallas import tpu as pltpu
from jax.experimental.pallas import tpu_sc as plsc

N_ENVS = 100
N_CLASSES = 1000
Z = 64
BATCH = 4096

_info = plsc.get_sparse_core_info()
_NC, _NS, _L = _info.num_cores, _info.num_subcores, _info.num_lanes
_NW = _NC * _NS
_BPW = BATCH // _NW
_NCH = _BPW // _L


def _sc_gather_body(y_hbm, e_hbm, mu_c_hbm, cov_c_hbm, mu_s_hbm, cov_s_hbm,
                    muc_out, mus_out, covc_out, covs_out,
                    y_v, e_v, muc_tab, covc_tab,
                    msl0, csl0, msl1, csl1,
                    mucb0, musb0, covcb0, covsb0,
                    mucb1, musb1, covcb1, covsb1,
                    sem_s0, sem_s1, sem_o0, sem_o1):
    wid = lax.axis_index("s") * _NC + lax.axis_index("c")
    base = wid * _BPW
    pltpu.sync_copy(y_hbm.at[pl.ds(base, _BPW)], y_v)
    pltpu.sync_copy(e_hbm.at[pl.ds(base, _BPW)], e_v)
    pltpu.sync_copy(mu_c_hbm, muc_tab)
    pltpu.sync_copy(cov_c_hbm, covc_tab)
    lanes = lax.iota(jnp.int32, _L)
    zeros = jnp.zeros((_L,), jnp.int32)
    slabs = ((msl0, csl0, sem_s0), (msl1, csl1, sem_s1))
    outs = ((mucb0, musb0, covcb0, covsb0, sem_o0),
            (mucb1, musb1, covcb1, covsb1, sem_o1))

    def _lane_scalar(vec, lane):
        return jnp.sum(jnp.where(lanes == lane, vec, zeros))

    def _fire_chunk(k, par):
        msl, csl, sem = slabs[par]
        y16 = y_v[pl.ds(k * _L, _L)]
        e16 = e_v[pl.ds(k * _L, _L)]
        cs = []
        for lane in range(_L):
            y_s = _lane_scalar(y16, lane)
            e_s = _lane_scalar(e16, lane)
            m8 = pl.multiple_of((e_s // 8) * 8, 8)
            c1 = pltpu.make_async_copy(
                mu_s_hbm.at[y_s, pl.ds(m8, 8), :], msl.at[lane], sem)
            c2 = pltpu.make_async_copy(
                cov_s_hbm.at[y_s, pl.ds(m8, 8), :], csl.at[lane], sem)
            c1.start(); c2.start()
            cs.append(c1); cs.append(c2)
        return cs

    def _out_copies(k, par):
        mucb, musb, covcb, covsb, sem = outs[par]
        row = pl.ds(base + k * _L, _L)
        return [pltpu.make_async_copy(mucb, muc_out.at[row], sem),
                pltpu.make_async_copy(musb, mus_out.at[row], sem),
                pltpu.make_async_copy(covcb, covc_out.at[row], sem),
                pltpu.make_async_copy(covsb, covs_out.at[row], sem)]

    def _extract_chunk(k, par, slab_copies, first):
        msl, csl, _ = slabs[par]
        mucb, musb, covcb, covsb, _ = outs[par]
        for c in slab_copies:
            c.wait()

        @pl.when(jnp.logical_not(first))
        def _():
            for c in _out_copies(k - 2, par):
                c.wait()

        e16 = e_v[pl.ds(k * _L, _L)]
        for lane in range(_L):
            e_s = _lane_scalar(e16, lane)
            off = lax.rem(e_s, 8)
            for j in range(Z // _L):
                sl = pl.ds(j * _L, _L)
                musb[lane, sl] = msl[lane, off, sl]
                covsb[lane, sl] = csl[lane, off, sl]
                mucb[lane, sl] = muc_tab[e_s, sl]
                covcb[lane, sl] = covc_tab[e_s, sl]
        for c in _out_copies(k, par):
            c.start()

    def _pair(p, _):
        a = 2 * p
        b = a + 1
        cs_a = _fire_chunk(a, 0)
        cs_b = _fire_chunk(b, 1)
        _extract_chunk(a, 0, cs_a, p == 0)
        _extract_chunk(b, 1, cs_b, p == 0)
        return 0

    lax.fori_loop(0, _NCH // 2, _pair, 0)
    for c in _out_copies(_NCH - 2, 0):
        c.wait()
    for c in _out_copies(_NCH - 1, 1):
        c.wait()


_sc_gather = functools.partial(
    pl.kernel,
    mesh=plsc.VectorSubcoreMesh(core_axis_name="c", subcore_axis_name="s"),
    out_type=[jax.ShapeDtypeStruct((BATCH, Z), jnp.float32)] * 4,
    scratch_types=(
        [pltpu.VMEM((_BPW,), jnp.int32)] * 2
        + [pltpu.VMEM((N_ENVS, Z), jnp.float32)] * 2
        + [pltpu.VMEM((_L, 8, Z), jnp.float32)] * 4
        + [pltpu.VMEM((_L, Z), jnp.float32)] * 8
        + [pltpu.SemaphoreType.DMA] * 4
    ),
    compiler_params=pltpu.CompilerParams(use_tc_tiling_on_sc=True,
                                         needs_layout_passes=False),
)(_sc_gather_body)


_BB = 256


def _tc_body(muc_ref, mus_ref, covc_ref, covs_ref, mu_ref, out_ref):
    mu_ref[...] = jnp.concatenate([muc_ref[...], mus_ref[...]], axis=-1)
    cov = jax.nn.softplus(
        jnp.concatenate([covc_ref[...], covs_ref[...]], axis=-1))
    eye = (lax.broadcasted_iota(jnp.int32, (2 * Z, 2 * Z), 0)
           == lax.broadcasted_iota(jnp.int32, (2 * Z, 2 * Z), 1))
    out_ref[...] = jnp.where(eye[None], cov[:, :, None], jnp.float32(0.0))


def _tc_diag(muc, mus, covc, covs):
    half = pl.BlockSpec((_BB, Z), lambda b: (b, 0))
    return pl.pallas_call(
        _tc_body,
        grid=(BATCH // _BB,),
        in_specs=[half, half, half, half],
        out_specs=[
            pl.BlockSpec((_BB, 2 * Z), lambda b: (b, 0)),
            pl.BlockSpec((_BB, 2 * Z, 2 * Z), lambda b: (b, 0, 0)),
        ],
        out_shape=[
            jax.ShapeDtypeStruct((BATCH, 2 * Z), jnp.float32),
            jax.ShapeDtypeStruct((BATCH, 2 * Z, 2 * Z), jnp.float32),
        ],
    )(muc, mus, covc, covs)


def kernel(y, e, mu_causal, cov_causal, mu_spurious, cov_spurious):
    y_flat = y[:, 0].astype(jnp.int32)
    e_flat = e[:, 0].astype(jnp.int32)
    muc, mus, covc, covs = _sc_gather(y_flat, e_flat, mu_causal, cov_causal,
                                      mu_spurious, cov_spurious)
    mu, cov_mat = _tc_diag(muc, mus, covc, covs)
    return mu, cov_mat


# speedup vs baseline: 1.0088x; 1.0088x over previous
import functools

import jax
import jax.numpy as jnp
from jax import lax
from jax.experimental import pallas as pl
from jax.experimental.pallas import tpu as pltpu
from jax.experimental.pallas import tpu_sc as plsc

N_ENVS = 100
N_CLASSES = 1000
Z = 64
BATCH = 4096

_info = plsc.get_sparse_core_info()
_NC, _NS, _L = _info.num_cores, _info.num_subcores, _info.num_lanes
_NW = _NC * _NS
_BPW = BATCH // _NW
_NCH = _BPW // _L


def _sc_gather_body(y_hbm, e_hbm, mu_c_hbm, cov_c_hbm, mu_s_hbm, cov_s_hbm,
                    muc_out, mus_out, covc_out, covs_out,
                    y_v, e_v, muc_tab, covc_tab, msl, csl,
                    muc_v, mus_v, covc_v, covs_v, sem):
    wid = lax.axis_index("s") * _NC + lax.axis_index("c")
    base = wid * _BPW
    pltpu.sync_copy(y_hbm.at[pl.ds(base, _BPW)], y_v)
    pltpu.sync_copy(e_hbm.at[pl.ds(base, _BPW)], e_v)
    pltpu.sync_copy(mu_c_hbm, muc_tab)
    pltpu.sync_copy(cov_c_hbm, covc_tab)
    lanes = lax.iota(jnp.int32, _L)
    zeros = jnp.zeros((_L,), jnp.int32)

    def _chunk(k, _):
        y16 = y_v[pl.ds(k * _L, _L)]
        e16 = e_v[pl.ds(k * _L, _L)]
        copies = []
        for lane in range(_L):
            y_s = jnp.sum(jnp.where(lanes == lane, y16, zeros))
            e_s = jnp.sum(jnp.where(lanes == lane, e16, zeros))
            m8 = pl.multiple_of((e_s // 8) * 8, 8)
            c1 = pltpu.make_async_copy(
                mu_s_hbm.at[y_s, pl.ds(m8, 8), :], msl.at[lane], sem)
            c2 = pltpu.make_async_copy(
                cov_s_hbm.at[y_s, pl.ds(m8, 8), :], csl.at[lane], sem)
            c1.start(); c2.start()
            copies.append(c1); copies.append(c2)
        for c in copies:
            c.wait()
        for lane in range(_L):
            e_s = jnp.sum(jnp.where(lanes == lane, e16, zeros))
            off = lax.rem(e_s, 8)
            r = k * _L + lane
            for j in range(Z // _L):
                sl = pl.ds(j * _L, _L)
                mus_v[r, sl] = msl[lane, off, sl]
                covs_v[r, sl] = csl[lane, off, sl]
                muc_v[r, sl] = muc_tab[e_s, sl]
                covc_v[r, sl] = covc_tab[e_s, sl]
        return 0

    lax.fori_loop(0, _NCH, _chunk, 0)
    rows = pl.ds(base, _BPW)
    outs = [pltpu.make_async_copy(muc_v, muc_out.at[rows], sem),
            pltpu.make_async_copy(mus_v, mus_out.at[rows], sem),
            pltpu.make_async_copy(covc_v, covc_out.at[rows], sem),
            pltpu.make_async_copy(covs_v, covs_out.at[rows], sem)]
    for c in outs:
        c.start()
    for c in outs:
        c.wait()


_sc_gather = functools.partial(
    pl.kernel,
    mesh=plsc.VectorSubcoreMesh(core_axis_name="c", subcore_axis_name="s"),
    out_type=[jax.ShapeDtypeStruct((BATCH, Z), jnp.float32)] * 4,
    scratch_types=[
        pltpu.VMEM((_BPW,), jnp.int32),
        pltpu.VMEM((_BPW,), jnp.int32),
        pltpu.VMEM((N_ENVS, Z), jnp.float32),
        pltpu.VMEM((N_ENVS, Z), jnp.float32),
        pltpu.VMEM((_L, 8, Z), jnp.float32),
        pltpu.VMEM((_L, 8, Z), jnp.float32),
        pltpu.VMEM((_BPW, Z), jnp.float32),
        pltpu.VMEM((_BPW, Z), jnp.float32),
        pltpu.VMEM((_BPW, Z), jnp.float32),
        pltpu.VMEM((_BPW, Z), jnp.float32),
        pltpu.SemaphoreType.DMA,
    ],
    compiler_params=pltpu.CompilerParams(use_tc_tiling_on_sc=True,
                                         needs_layout_passes=False),
)(_sc_gather_body)


_BB = 256


def _tc_body(muc_ref, mus_ref, covc_ref, covs_ref, mu_ref, out_ref):
    mu_ref[...] = jnp.concatenate([muc_ref[...], mus_ref[...]], axis=-1)
    cov = jax.nn.softplus(
        jnp.concatenate([covc_ref[...], covs_ref[...]], axis=-1))
    eye = (lax.broadcasted_iota(jnp.int32, (2 * Z, 2 * Z), 0)
           == lax.broadcasted_iota(jnp.int32, (2 * Z, 2 * Z), 1))
    out_ref[...] = jnp.where(eye[None], cov[:, :, None], jnp.float32(0.0))


def _tc_diag(muc, mus, covc, covs):
    half = pl.BlockSpec((_BB, Z), lambda b: (b, 0))
    return pl.pallas_call(
        _tc_body,
        grid=(BATCH // _BB,),
        in_specs=[half, half, half, half],
        out_specs=[
            pl.BlockSpec((_BB, 2 * Z), lambda b: (b, 0)),
            pl.BlockSpec((_BB, 2 * Z, 2 * Z), lambda b: (b, 0, 0)),
        ],
        out_shape=[
            jax.ShapeDtypeStruct((BATCH, 2 * Z), jnp.float32),
            jax.ShapeDtypeStruct((BATCH, 2 * Z, 2 * Z), jnp.float32),
        ],
    )(muc, mus, covc, covs)


def kernel(y, e, mu_causal, cov_causal, mu_spurious, cov_spurious):
    y_flat = y[:, 0].astype(jnp.int32)
    e_flat = e[:, 0].astype(jnp.int32)
    muc, mus, covc, covs = _sc_gather(y_flat, e_flat, mu_causal, cov_causal,
                                      mu_spurious, cov_spurious)
    mu, cov_mat = _tc_diag(muc, mus, covc, covs)
    return mu, cov_mat


# speedup vs baseline: 1.0249x; 1.0160x over previous
import functools

import jax
import jax.numpy as jnp
from jax import lax
from jax.experimental import pallas as pl
from jax.experimental.pallas import tpu as pltpu
from jax.experimental.pallas import tpu_sc as plsc

N_ENVS = 100
N_CLASSES = 1000
Z = 64
BATCH = 4096

_info = plsc.get_sparse_core_info()
_NC, _NS, _L = _info.num_cores, _info.num_subcores, _info.num_lanes
_NW = _NC * _NS
_BPW = BATCH // _NW
_NCH = _BPW // _L


def _sc_gather_body(y_hbm, e_hbm, mu_c_hbm, cov_c_hbm, mu_s_hbm, cov_s_hbm,
                    mu_out, cov_out,
                    y_v, e_v, muc_tab, covc_tab, msl, csl,
                    mu_v, cov_v, sem):
    wid = lax.axis_index("s") * _NC + lax.axis_index("c")
    base = wid * _BPW
    pltpu.sync_copy(y_hbm.at[pl.ds(base, _BPW)], y_v)
    pltpu.sync_copy(e_hbm.at[pl.ds(base, _BPW)], e_v)
    pltpu.sync_copy(mu_c_hbm, muc_tab)
    pltpu.sync_copy(cov_c_hbm, covc_tab)
    lanes = lax.iota(jnp.int32, _L)
    zeros = jnp.zeros((_L,), jnp.int32)

    def _chunk(k, _):
        y16 = y_v[pl.ds(k * _L, _L)]
        e16 = e_v[pl.ds(k * _L, _L)]
        copies = []
        for lane in range(_L):
            y_s = jnp.sum(jnp.where(lanes == lane, y16, zeros))
            e_s = jnp.sum(jnp.where(lanes == lane, e16, zeros))
            m8 = pl.multiple_of((e_s // 8) * 8, 8)
            c1 = pltpu.make_async_copy(
                mu_s_hbm.at[y_s, pl.ds(m8, 8), :], msl.at[lane], sem)
            c2 = pltpu.make_async_copy(
                cov_s_hbm.at[y_s, pl.ds(m8, 8), :], csl.at[lane], sem)
            c1.start(); c2.start()
            copies.append(c1); copies.append(c2)
        for c in copies:
            c.wait()
        for lane in range(_L):
            e_s = jnp.sum(jnp.where(lanes == lane, e16, zeros))
            off = lax.rem(e_s, 8)
            r = k * _L + lane
            for j in range(Z // _L):
                sl = pl.ds(j * _L, _L)
                sh = pl.ds(Z + j * _L, _L)
                mu_v[r, sl] = muc_tab[e_s, sl]
                mu_v[r, sh] = msl[lane, off, sl]
                cov_v[r, sl] = covc_tab[e_s, sl]
                cov_v[r, sh] = csl[lane, off, sl]
        return 0

    lax.fori_loop(0, _NCH, _chunk, 0)
    rows = pl.ds(base, _BPW)
    outs = [pltpu.make_async_copy(mu_v, mu_out.at[rows], sem),
            pltpu.make_async_copy(cov_v, cov_out.at[rows], sem)]
    for c in outs:
        c.start()
    for c in outs:
        c.wait()


_sc_gather = functools.partial(
    pl.kernel,
    mesh=plsc.VectorSubcoreMesh(core_axis_name="c", subcore_axis_name="s"),
    out_type=[jax.ShapeDtypeStruct((BATCH, 2 * Z), jnp.float32)] * 2,
    scratch_types=[
        pltpu.VMEM((_BPW,), jnp.int32),
        pltpu.VMEM((_BPW,), jnp.int32),
        pltpu.VMEM((N_ENVS, Z), jnp.float32),
        pltpu.VMEM((N_ENVS, Z), jnp.float32),
        pltpu.VMEM((_L, 8, Z), jnp.float32),
        pltpu.VMEM((_L, 8, Z), jnp.float32),
        pltpu.VMEM((_BPW, 2 * Z), jnp.float32),
        pltpu.VMEM((_BPW, 2 * Z), jnp.float32),
        pltpu.SemaphoreType.DMA,
    ],
    compiler_params=pltpu.CompilerParams(use_tc_tiling_on_sc=True,
                                         needs_layout_passes=False),
)(_sc_gather_body)


_BB = 256


def _tc_body(cov_ref, out_ref):
    cov = jax.nn.softplus(cov_ref[...])
    eye = (lax.broadcasted_iota(jnp.int32, (2 * Z, 2 * Z), 0)
           == lax.broadcasted_iota(jnp.int32, (2 * Z, 2 * Z), 1))
    out_ref[...] = jnp.where(eye[None], cov[:, :, None], jnp.float32(0.0))


def _tc_diag(cov_cat):
    return pl.pallas_call(
        _tc_body,
        grid=(BATCH // _BB,),
        in_specs=[pl.BlockSpec((_BB, 2 * Z), lambda b: (b, 0))],
        out_specs=pl.BlockSpec((_BB, 2 * Z, 2 * Z), lambda b: (b, 0, 0)),
        out_shape=jax.ShapeDtypeStruct((BATCH, 2 * Z, 2 * Z), jnp.float32),
    )(cov_cat)


def kernel(y, e, mu_causal, cov_causal, mu_spurious, cov_spurious):
    y_flat = y[:, 0].astype(jnp.int32)
    e_flat = e[:, 0].astype(jnp.int32)
    mu, cov_cat = _sc_gather(y_flat, e_flat, mu_causal, cov_causal,
                             mu_spurious, cov_spurious)
    cov_mat = _tc_diag(cov_cat)
    return mu, cov_mat
